# Initial kernel scaffold; baseline (speedup 1.0000x reference)
import jax
import jax.numpy as jnp
from jax.experimental import pallas as pl
from jax.experimental.pallas import tpu as pltpu

_H = 128
_W = 128
_BIG = jnp.float32(1e9)


def _minplus(x):
    n = x.shape[0]
    o_idx = jax.lax.broadcasted_iota(jnp.float32, x.shape, 0)

    def body(s, y):
        row = jax.lax.dynamic_slice_in_dim(x, s, 1, axis=0)
        d = o_idx - s.astype(jnp.float32)
        return jnp.minimum(y, row + d * d)

    y0 = jnp.full(x.shape, _BIG, jnp.float32)
    return jax.lax.fori_loop(0, n, body, y0)


def _dist2_T(mask):
    x0 = jnp.where(mask, jnp.float32(0.0), _BIG)
    g2 = _minplus(x0)
    return _minplus(g2.T)


def _loss_kernel(pred_ref, targ_ref, out_ref):
    total = jnp.float32(0.0)
    thr = jnp.float32(0.3) + jnp.float32(1e-5) * jnp.float32(1.0)
    for i in range(pred_ref.shape[0]):
        pred = pred_ref[i]
        targ = targ_ref[i]
        pm = jnp.abs(pred - jnp.float32(1.0)) <= thr
        tg = targ != jnp.float32(0.0)

        d2t_tgt = _dist2_T(tg)
        d2t_pred = _dist2_T(pm)

        n_pred = jnp.sum(pm.astype(jnp.float32))
        n_gt = jnp.sum(tg.astype(jnp.float32))
        s1 = jnp.sum(jnp.where(pm.T, jnp.sqrt(d2t_tgt), jnp.float32(0.0)))
        s2 = jnp.sum(jnp.where(tg.T, jnp.sqrt(d2t_pred), jnp.float32(0.0)))
        term = (s1 + s2) / (jnp.float32(2.0) * n_gt)
        term = jnp.where((n_pred == 0.0) | (n_gt == 0.0), jnp.float32(0.0), term)
        total = total + term
    out_ref[0, 0] = total / jnp.float32(pred_ref.shape[0])


def kernel(pred, target):
    n = pred.shape[0] * pred.shape[1]
    pred3 = pred.reshape(n, _H, _W)
    targ3 = target.reshape(n, _H, _W)
    out = pl.pallas_call(
        _loss_kernel,
        out_shape=jax.ShapeDtypeStruct((1, 1), jnp.float32),
        in_specs=[
            pl.BlockSpec(memory_space=pltpu.VMEM),
            pl.BlockSpec(memory_space=pltpu.VMEM),
        ],
        out_specs=pl.BlockSpec(memory_space=pltpu.SMEM),
    )(pred3, targ3)
    return out[0, 0]


# speedup vs baseline: 241.1372x; 241.1372x over previous
import jax
import jax.numpy as jnp
import numpy as np
from jax.experimental import pallas as pl
from jax.experimental.pallas import tpu as pltpu

_H = 128
_W = 128
_BIG = np.float32(1e9)


def _minplus(x_ref):
    n = x_ref.shape[0]
    o_idx = jax.lax.broadcasted_iota(jnp.int32, x_ref.shape, 0).astype(jnp.float32)

    def body(s, y):
        row = x_ref[pl.ds(s, 1), :]
        d = o_idx - s.astype(jnp.float32)
        return jnp.minimum(y, row + d * d)

    y0 = jnp.full(x_ref.shape, _BIG, jnp.float32)
    return jax.lax.fori_loop(0, n, body, y0)


def _dist2_T(mask, x_ref):
    x_ref[...] = jnp.where(mask, jnp.float32(0.0), _BIG)
    g2 = _minplus(x_ref)
    x_ref[...] = g2.T
    return _minplus(x_ref)


def _loss_kernel(pred_ref, targ_ref, out_ref, x_ref):
    total = jnp.float32(0.0)
    thr = jnp.float32(0.3) + jnp.float32(1e-5) * jnp.float32(1.0)
    for i in range(pred_ref.shape[0]):
        pred = pred_ref[i]
        targ = targ_ref[i]
        pm = jnp.abs(pred - jnp.float32(1.0)) <= thr
        tg = targ != jnp.float32(0.0)

        d2t_tgt = _dist2_T(tg, x_ref)
        d2t_pred = _dist2_T(pm, x_ref)

        n_pred = jnp.sum(pm.astype(jnp.float32))
        n_gt = jnp.sum(tg.astype(jnp.float32))
        s1 = jnp.sum(jnp.where(pm.T, jnp.sqrt(d2t_tgt), jnp.float32(0.0)))
        s2 = jnp.sum(jnp.where(tg.T, jnp.sqrt(d2t_pred), jnp.float32(0.0)))
        term = (s1 + s2) / (jnp.float32(2.0) * n_gt)
        term = jnp.where((n_pred == 0.0) | (n_gt == 0.0), jnp.float32(0.0), term)
        total = total + term
    out_ref[0, 0] = total / jnp.float32(pred_ref.shape[0])


def kernel(pred, target):
    n = pred.shape[0] * pred.shape[1]
    pred3 = pred.reshape(n, _H, _W)
    targ3 = target.reshape(n, _H, _W)
    out = pl.pallas_call(
        _loss_kernel,
        out_shape=jax.ShapeDtypeStruct((1, 1), jnp.float32),
        in_specs=[
            pl.BlockSpec(memory_space=pltpu.VMEM),
            pl.BlockSpec(memory_space=pltpu.VMEM),
        ],
        out_specs=pl.BlockSpec(memory_space=pltpu.SMEM),
        scratch_shapes=[pltpu.VMEM((_H, _W), jnp.float32)],
    )(pred3, targ3)
    return out[0, 0]


# speedup vs baseline: 707.9344x; 2.9358x over previous
import jax
import jax.numpy as jnp
import numpy as np
from jax.experimental import pallas as pl
from jax.experimental.pallas import tpu as pltpu

_H = 128
_W = 128
_K = 8
_BIG = np.float32(1e9)


def _loss_kernel(pred_ref, targ_ref, predT_ref, targT_ref, out_ref,
                 x_ref, g_ref, t_ref, d_ref):
    thr = jnp.float32(0.3) + jnp.float32(1e-5) * jnp.float32(1.0)

    for k in range(_K):
        pm = jnp.abs(pred_ref[k] - jnp.float32(1.0)) <= thr
        tg = targ_ref[k] != jnp.float32(0.0)
        x_ref[:, k * _W:(k + 1) * _W] = jnp.where(pm, jnp.float32(0.0), _BIG)
        x_ref[:, (_K + k) * _W:(_K + k + 1) * _W] = jnp.where(
            tg, jnp.float32(0.0), _BIG)

    g_ref[0:1, :] = x_ref[0:1, :]

    def fwd(r, g):
        g = jnp.minimum(x_ref[pl.ds(r, 1), :], g + jnp.float32(1.0))
        g_ref[pl.ds(r, 1), :] = g
        return g

    jax.lax.fori_loop(1, _H, fwd, x_ref[0:1, :])

    def bwd(rr, b):
        r = _H - 2 - rr
        b = jnp.minimum(g_ref[pl.ds(r, 1), :], b + jnp.float32(1.0))
        g_ref[pl.ds(r, 1), :] = b
        return b

    jax.lax.fori_loop(0, _H - 1, bwd, g_ref[_H - 1:_H, :])

    for k in range(2 * _K):
        blk = g_ref[:, k * _W:(k + 1) * _W]
        t_ref[:, k * _W:(k + 1) * _W] = (blk * blk).T

    o_col = jax.lax.broadcasted_iota(jnp.int32, (_H, 1), 0).astype(jnp.float32)
    d_ref[...] = t_ref[0:1, :] + o_col * o_col

    def p2(s, carry):
        row = t_ref[pl.ds(s, 1), :]
        d = o_col - s.astype(jnp.float32)
        d_ref[...] = jnp.minimum(d_ref[...], row + d * d)
        return carry

    jax.lax.fori_loop(1, _H, p2, jnp.int32(0))

    total = jnp.float32(0.0)
    for k in range(_K):
        pmT = jnp.abs(predT_ref[k] - jnp.float32(1.0)) <= thr
        tgT = targT_ref[k] != jnp.float32(0.0)
        d2_pred_T = d_ref[:, k * _W:(k + 1) * _W]
        d2_tgt_T = d_ref[:, (_K + k) * _W:(_K + k + 1) * _W]
        n_pred = jnp.sum(pmT.astype(jnp.float32))
        n_gt = jnp.sum(tgT.astype(jnp.float32))
        s1 = jnp.sum(jnp.where(pmT, jnp.sqrt(d2_tgt_T), jnp.float32(0.0)))
        s2 = jnp.sum(jnp.where(tgT, jnp.sqrt(d2_pred_T), jnp.float32(0.0)))
        term = (s1 + s2) / (jnp.float32(2.0) * n_gt)
        term = jnp.where((n_pred == 0.0) | (n_gt == 0.0), jnp.float32(0.0),
                         term)
        total = total + term
    out_ref[0, 0] = total / jnp.float32(_K)


def kernel(pred, target):
    n = pred.shape[0] * pred.shape[1]
    pred3 = pred.reshape(n, _H, _W)
    targ3 = target.reshape(n, _H, _W)
    predT = jnp.swapaxes(pred3, 1, 2)
    targT = jnp.swapaxes(targ3, 1, 2)
    out = pl.pallas_call(
        _loss_kernel,
        out_shape=jax.ShapeDtypeStruct((1, 1), jnp.float32),
        in_specs=[pl.BlockSpec(memory_space=pltpu.VMEM)] * 4,
        out_specs=pl.BlockSpec(memory_space=pltpu.SMEM),
        scratch_shapes=[
            pltpu.VMEM((_H, 2 * _K * _W), jnp.float32),
            pltpu.VMEM((_H, 2 * _K * _W), jnp.float32),
            pltpu.VMEM((_H, 2 * _K * _W), jnp.float32),
            pltpu.VMEM((_H, 2 * _K * _W), jnp.float32),
        ],
    )(pred3, targ3, predT, targT)
    return out[0, 0]


# speedup vs baseline: 928.3852x; 1.3114x over previous
import jax
import jax.numpy as jnp
import numpy as np
from jax.experimental import pallas as pl
from jax.experimental.pallas import tpu as pltpu

_H = 128
_W = 128
_K = 8
_BIG = np.float32(1e9)


def _loss_kernel(pred_ref, targ_ref, out_ref, x_ref, g_ref, t_ref, d_ref):
    thr = jnp.float32(0.3) + jnp.float32(1e-5) * jnp.float32(1.0)

    def masks(k):
        pm = jnp.abs(pred_ref[k] - jnp.float32(1.0)) <= thr
        tg = targ_ref[k] != jnp.float32(0.0)
        return pm, tg

    for k in range(_K):
        pm, tg = masks(k)
        x_ref[:, k * _W:(k + 1) * _W] = jnp.where(
            pm, jnp.float32(0.0), _BIG).T
        x_ref[:, (_K + k) * _W:(_K + k + 1) * _W] = jnp.where(
            tg, jnp.float32(0.0), _BIG).T

    g_ref[0:1, :] = x_ref[0:1, :]

    def fwd(r, g):
        g = jnp.minimum(x_ref[pl.ds(r, 1), :], g + jnp.float32(1.0))
        g_ref[pl.ds(r, 1), :] = g
        return g

    jax.lax.fori_loop(1, _H, fwd, x_ref[0:1, :])

    def bwd(rr, b):
        r = _H - 2 - rr
        b = jnp.minimum(g_ref[pl.ds(r, 1), :], b + jnp.float32(1.0))
        g_ref[pl.ds(r, 1), :] = b
        return b

    jax.lax.fori_loop(0, _H - 1, bwd, g_ref[_H - 1:_H, :])

    for k in range(2 * _K):
        blk = g_ref[:, k * _W:(k + 1) * _W]
        t_ref[:, k * _W:(k + 1) * _W] = (blk * blk).T

    o_col = jax.lax.broadcasted_iota(jnp.int32, (_H, 1), 0).astype(jnp.float32)
    d_ref[...] = jnp.minimum(
        t_ref[0:1, :] + o_col * o_col,
        t_ref[1:2, :] + (o_col - 1.0) * (o_col - 1.0))

    def p2(u, carry):
        s = 2 * u
        row0 = t_ref[pl.ds(s, 1), :]
        row1 = t_ref[pl.ds(s + 1, 1), :]
        d0 = o_col - s.astype(jnp.float32)
        d1 = d0 - jnp.float32(1.0)
        cand = jnp.minimum(row0 + d0 * d0, row1 + d1 * d1)
        d_ref[...] = jnp.minimum(d_ref[...], cand)
        return carry

    jax.lax.fori_loop(1, _H // 2, p2, jnp.int32(0))

    total = jnp.float32(0.0)
    for k in range(_K):
        pm, tg = masks(k)
        d2_pred = d_ref[:, k * _W:(k + 1) * _W]
        d2_tgt = d_ref[:, (_K + k) * _W:(_K + k + 1) * _W]
        n_pred = jnp.sum(pm.astype(jnp.float32))
        n_gt = jnp.sum(tg.astype(jnp.float32))
        s1 = jnp.sum(jnp.where(pm, jnp.sqrt(d2_tgt), jnp.float32(0.0)))
        s2 = jnp.sum(jnp.where(tg, jnp.sqrt(d2_pred), jnp.float32(0.0)))
        term = (s1 + s2) / (jnp.float32(2.0) * n_gt)
        term = jnp.where((n_pred == 0.0) | (n_gt == 0.0), jnp.float32(0.0),
                         term)
        total = total + term
    out_ref[0, 0] = total / jnp.float32(_K)


def kernel(pred, target):
    n = pred.shape[0] * pred.shape[1]
    pred3 = pred.reshape(n, _H, _W)
    targ3 = target.reshape(n, _H, _W)
    out = pl.pallas_call(
        _loss_kernel,
        out_shape=jax.ShapeDtypeStruct((1, 1), jnp.float32),
        in_specs=[pl.BlockSpec(memory_space=pltpu.VMEM)] * 2,
        out_specs=pl.BlockSpec(memory_space=pltpu.SMEM),
        scratch_shapes=[
            pltpu.VMEM((_H, 2 * _K * _W), jnp.float32),
            pltpu.VMEM((_H, 2 * _K * _W), jnp.float32),
            pltpu.VMEM((_H, 2 * _K * _W), jnp.float32),
            pltpu.VMEM((_H, 2 * _K * _W), jnp.float32),
        ],
    )(pred3, targ3)
    return out[0, 0]


# speedup vs baseline: 959.9491x; 1.0340x over previous
import jax
import jax.numpy as jnp
import numpy as np
from jax.experimental import pallas as pl
from jax.experimental.pallas import tpu as pltpu

_H = 128
_W = 128
_K = 8
_BIG = np.float32(1e9)


def _loss_kernel(pred_ref, targ_ref, out_ref, x_ref, g_ref, t_ref, d_ref):
    thr = jnp.float32(0.3) + jnp.float32(1e-5) * jnp.float32(1.0)

    def masks(k):
        pm = jnp.abs(pred_ref[k] - jnp.float32(1.0)) <= thr
        tg = targ_ref[k] != jnp.float32(0.0)
        return pm, tg

    for k in range(_K):
        pm, tg = masks(k)
        x_ref[:, k * _W:(k + 1) * _W] = jnp.where(
            pm, jnp.float32(0.0), _BIG).T
        x_ref[:, (_K + k) * _W:(_K + k + 1) * _W] = jnp.where(
            tg, jnp.float32(0.0), _BIG).T

    g_ref[0:1, :] = x_ref[0:1, :]

    def fwd(r, g):
        g = jnp.minimum(x_ref[pl.ds(r, 1), :], g + jnp.float32(1.0))
        g_ref[pl.ds(r, 1), :] = g
        return g

    jax.lax.fori_loop(1, _H, fwd, x_ref[0:1, :])

    def bwd(rr, b):
        r = _H - 2 - rr
        b = jnp.minimum(g_ref[pl.ds(r, 1), :], b + jnp.float32(1.0))
        g_ref[pl.ds(r, 1), :] = b
        return b

    jax.lax.fori_loop(0, _H - 1, bwd, g_ref[_H - 1:_H, :])

    for k in range(2 * _K):
        blk = g_ref[:, k * _W:(k + 1) * _W]
        t_ref[:, k * _W:(k + 1) * _W] = (blk * blk).T

    o_col = jax.lax.broadcasted_iota(jnp.int32, (_H, 1), 0).astype(jnp.float32)

    def cand4(s_f32, base):
        c = None
        for q in range(4):
            d = o_col - (s_f32 + jnp.float32(q))
            term = t_ref[pl.ds(base + q, 1), :] + d * d
            c = term if c is None else jnp.minimum(c, term)
        return c

    d_ref[...] = cand4(jnp.float32(0.0), 0)

    def p2(u, carry):
        s = 4 * u
        d_ref[...] = jnp.minimum(d_ref[...], cand4(s.astype(jnp.float32), s))
        return carry

    jax.lax.fori_loop(1, _H // 4, p2, jnp.int32(0))

    total = jnp.float32(0.0)
    for k in range(_K):
        pm, tg = masks(k)
        d2_pred = d_ref[:, k * _W:(k + 1) * _W]
        d2_tgt = d_ref[:, (_K + k) * _W:(_K + k + 1) * _W]
        n_pred = jnp.sum(pm.astype(jnp.float32))
        n_gt = jnp.sum(tg.astype(jnp.float32))
        s1 = jnp.sum(jnp.where(pm, jnp.sqrt(d2_tgt), jnp.float32(0.0)))
        s2 = jnp.sum(jnp.where(tg, jnp.sqrt(d2_pred), jnp.float32(0.0)))
        term = (s1 + s2) / (jnp.float32(2.0) * n_gt)
        term = jnp.where((n_pred == 0.0) | (n_gt == 0.0), jnp.float32(0.0),
                         term)
        total = total + term
    out_ref[0, 0] = total / jnp.float32(_K)


def kernel(pred, target):
    n = pred.shape[0] * pred.shape[1]
    pred3 = pred.reshape(n, _H, _W)
    targ3 = target.reshape(n, _H, _W)
    out = pl.pallas_call(
        _loss_kernel,
        out_shape=jax.ShapeDtypeStruct((1, 1), jnp.float32),
        in_specs=[pl.BlockSpec(memory_space=pltpu.VMEM)] * 2,
        out_specs=pl.BlockSpec(memory_space=pltpu.SMEM),
        scratch_shapes=[
            pltpu.VMEM((_H, 2 * _K * _W), jnp.float32),
            pltpu.VMEM((_H, 2 * _K * _W), jnp.float32),
            pltpu.VMEM((_H, 2 * _K * _W), jnp.float32),
            pltpu.VMEM((_H, 2 * _K * _W), jnp.float32),
        ],
    )(pred3, targ3)
    return out[0, 0]


# speedup vs baseline: 988.1427x; 1.0294x over previous
import jax
import jax.numpy as jnp
import numpy as np
from jax.experimental import pallas as pl
from jax.experimental.pallas import tpu as pltpu

_H = 128
_W = 128
_K = 8
_BIG = np.float32(1e9)


def _loss_kernel(pred_ref, targ_ref, out_ref, x_ref, g_ref, t_ref, d_ref):
    thr = jnp.float32(0.3) + jnp.float32(1e-5) * jnp.float32(1.0)

    def masks(k):
        pm = jnp.abs(pred_ref[k] - jnp.float32(1.0)) <= thr
        tg = targ_ref[k] != jnp.float32(0.0)
        return pm, tg

    for k in range(_K):
        pm, tg = masks(k)
        x_ref[:, k * _W:(k + 1) * _W] = jnp.where(
            pm, jnp.float32(0.0), _BIG).T
        x_ref[:, (_K + k) * _W:(_K + k + 1) * _W] = jnp.where(
            tg, jnp.float32(0.0), _BIG).T

    g_ref[0:1, :] = x_ref[0:1, :]

    def fwd(u, g):
        r = 2 * u + 1
        g = jnp.minimum(x_ref[pl.ds(r, 1), :], g + jnp.float32(1.0))
        g_ref[pl.ds(r, 1), :] = g
        g = jnp.minimum(x_ref[pl.ds(r + 1, 1), :], g + jnp.float32(1.0))
        g_ref[pl.ds(r + 1, 1), :] = g
        return g

    gl = jax.lax.fori_loop(0, (_H - 2) // 2, fwd, x_ref[0:1, :])
    b0 = jnp.minimum(x_ref[_H - 1:_H, :], gl + jnp.float32(1.0))
    g_ref[_H - 1:_H, :] = b0 * b0

    def bwd(u, b):
        r = _H - 2 - 2 * u
        b = jnp.minimum(g_ref[pl.ds(r, 1), :], b + jnp.float32(1.0))
        g_ref[pl.ds(r, 1), :] = b * b
        b2 = jnp.minimum(g_ref[pl.ds(r - 1, 1), :], b + jnp.float32(1.0))
        g_ref[pl.ds(r - 1, 1), :] = b2 * b2
        return b2

    blast = jax.lax.fori_loop(0, (_H - 2) // 2, bwd, b0)
    bfin = jnp.minimum(g_ref[0:1, :], blast + jnp.float32(1.0))
    g_ref[0:1, :] = bfin * bfin

    for k in range(2 * _K):
        blk = g_ref[:, k * _W:(k + 1) * _W]
        t_ref[:, k * _W:(k + 1) * _W] = blk.T

    o_col = jax.lax.broadcasted_iota(jnp.int32, (_H, 1), 0).astype(jnp.float32)

    def cand4(s_f32, base):
        c = None
        for q in range(4):
            d = o_col - (s_f32 + jnp.float32(q))
            term = t_ref[pl.ds(base + q, 1), :] + d * d
            c = term if c is None else jnp.minimum(c, term)
        return c

    d_ref[...] = cand4(jnp.float32(0.0), 0)

    def p2(u, carry):
        s = 4 * u
        d_ref[...] = jnp.minimum(d_ref[...], cand4(s.astype(jnp.float32), s))
        return carry

    jax.lax.fori_loop(1, _H // 4, p2, jnp.int32(0))

    total = jnp.float32(0.0)
    for k in range(_K):
        pm, tg = masks(k)
        d2_pred = d_ref[:, k * _W:(k + 1) * _W]
        d2_tgt = d_ref[:, (_K + k) * _W:(_K + k + 1) * _W]
        n_pred = jnp.sum(pm.astype(jnp.float32))
        n_gt = jnp.sum(tg.astype(jnp.float32))
        s1 = jnp.sum(jnp.where(pm, jnp.sqrt(d2_tgt), jnp.float32(0.0)))
        s2 = jnp.sum(jnp.where(tg, jnp.sqrt(d2_pred), jnp.float32(0.0)))
        term = (s1 + s2) / (jnp.float32(2.0) * n_gt)
        term = jnp.where((n_pred == 0.0) | (n_gt == 0.0), jnp.float32(0.0),
                         term)
        total = total + term
    out_ref[0, 0] = total / jnp.float32(_K)


def kernel(pred, target):
    n = pred.shape[0] * pred.shape[1]
    pred3 = pred.reshape(n, _H, _W)
    targ3 = target.reshape(n, _H, _W)
    out = pl.pallas_call(
        _loss_kernel,
        out_shape=jax.ShapeDtypeStruct((1, 1), jnp.float32),
        in_specs=[pl.BlockSpec(memory_space=pltpu.VMEM)] * 2,
        out_specs=pl.BlockSpec(memory_space=pltpu.SMEM),
        scratch_shapes=[
            pltpu.VMEM((_H, 2 * _K * _W), jnp.float32),
            pltpu.VMEM((_H, 2 * _K * _W), jnp.float32),
            pltpu.VMEM((_H, 2 * _K * _W), jnp.float32),
            pltpu.VMEM((_H, 2 * _K * _W), jnp.float32),
        ],
    )(pred3, targ3)
    return out[0, 0]
